# baseline (speedup 1.0000x reference)
import dataclasses
import functools
import math

import jax
import jax.numpy as jnp
from jax import lax
from jax.experimental import pallas as pl
from jax.experimental.pallas import tpu as pltpu
from jax.experimental.pallas import tpu_sc as plsc

POOLED = 7
NBINS = POOLED * POOLED
SCALE = 0.125
NPAD = 1024
T = 64


def _prep_body(n_real, H, W, rois_ref, idx_ref, w_ref):
    r = rois_ref[...]
    bidx = r[0:1, :].astype(jnp.int32)
    cx, cy = r[1:2, :], r[2:3, :]
    hh, ww = r[3:4, :], r[4:5, :]
    th = r[5:6, :] * (math.pi / 180.0)

    Sx = ww * (SCALE / POOLED)
    Sy = hh * (SCALE / POOLED)
    Al, Be = jnp.cos(th), jnp.sin(th)
    dx = dy = -POOLED / 2.0
    M00 = Al * Sx
    M01 = Be * Sy
    M02 = Al * Sx * dx + Be * Sy * dy + cx * SCALE
    M10 = -Be * Sx
    M11 = Al * Sy
    M12 = -Be * Sx * dx + Al * Sy * dy + cy * SCALE

    bi = lax.broadcasted_iota(jnp.int32, (NBINS, NPAD), 0)
    lane = lax.broadcasted_iota(jnp.int32, (NBINS, NPAD), 1)
    pwf = (bi % POOLED).astype(jnp.float32) + 0.5
    phf = (bi // POOLED).astype(jnp.float32) + 0.5
    Px = M00 * pwf + M01 * phf + M02
    Py = M10 * pwf + M11 * phf + M12

    vf = ((Px >= 0.0) & (Px <= W - 1.0) & (Py >= 0.0) & (Py <= H - 1.0)
          & (lane < n_real)).astype(jnp.float32)
    x0i = Px.astype(jnp.int32)
    y0i = Py.astype(jnp.int32)
    wx = Px - x0i.astype(jnp.float32)
    wy = Py - y0i.astype(jnp.float32)
    x0 = jnp.clip(x0i, 0, W - 1)
    x1 = jnp.clip(x0i + 1, 0, W - 1)
    y0 = jnp.clip(y0i, 0, H - 1)
    y1 = jnp.clip(y0i + 1, 0, H - 1)

    base = bidx * (H * W)
    idx_ref[0] = base + y0 * W + x0
    idx_ref[1] = base + y0 * W + x1
    idx_ref[2] = base + y1 * W + x0
    idx_ref[3] = base + y1 * W + x1
    w_ref[0] = (1.0 - wy) * (1.0 - wx) * vf
    w_ref[1] = (1.0 - wy) * wx * vf
    w_ref[2] = wy * (1.0 - wx) * vf
    w_ref[3] = wy * wx * vf


def _sc_pooled_rows(table, idx_g, w_g, C):
    G = idx_g.shape[0]
    K = G * T
    info = plsc.get_sparse_core_info()
    NC, NS = info.num_cores, info.num_subcores
    NW = NC * NS
    S = G // NW
    assert S * NW == G and S % 2 == 0 and S >= 4

    mesh = plsc.VectorSubcoreMesh(core_axis_name="core", subcore_axis_name="subcore")

    cp = pltpu.CompilerParams()
    if "needs_layout_passes" in pltpu.CompilerParams.__dataclass_fields__:
        cp = dataclasses.replace(cp, needs_layout_passes=False)

    @functools.partial(
        pl.kernel,
        out_type=jax.ShapeDtypeStruct((K, C), jnp.float32),
        mesh=mesh,
        scratch_types=(
            [pltpu.VMEM((T, C), jnp.float32) for _ in range(4)]
            + [pltpu.VMEM((T, C), jnp.float32) for _ in range(2)]
            + [pltpu.VMEM((T,), jnp.int32) for _ in range(8)]
            + [pltpu.VMEM((4 * T,), jnp.float32) for _ in range(2)]
            + [pltpu.SemaphoreType.DMA for _ in range(7)]
        ),
        compiler_params=cp,
    )
    def sc_kernel(table_hbm, idx_hbm, w_hbm, out_hbm,
                  r0, r1, r2, r3,
                  oa, ob, ia0, ia1, ia2, ia3, ib0, ib1, ib2, ib3, wa, wb,
                  sia, sib, srow, soa, sob, swa, swb):
        rows = (r0, r1, r2, r3)
        idxs = ((ia0, ia1, ia2, ia3), (ib0, ib1, ib2, ib3))
        outs, ws = (oa, ob), (wa, wb)
        isems, osems = (sia, sib), (soa, sob)
        wsems = (swa, swb)

        wid = lax.axis_index("subcore") * NC + lax.axis_index("core")
        base = wid * S

        def idx_start(g, slot):
            for c in range(4):
                pltpu.make_async_copy(idx_hbm.at[g, pl.ds(c * T, T)],
                                      idxs[slot][c], isems[slot]).start()

        def idx_wait(slot):
            for c in range(4):
                pltpu.make_async_copy(idx_hbm.at[0, pl.ds(c * T, T)],
                                      idxs[slot][c], isems[slot]).wait()

        def w_start(g, slot):
            pltpu.make_async_copy(w_hbm.at[g], ws[slot], wsems[slot]).start()

        def w_wait(slot):
            pltpu.make_async_copy(w_hbm.at[0], ws[slot], wsems[slot]).wait()

        def gathers_start(slot):
            for c in range(4):
                pltpu.make_async_copy(table_hbm.at[idxs[slot][c]],
                                      rows[c], srow).start()

        def gathers_wait(slot):
            for c in range(4):
                pltpu.make_async_copy(table_hbm.at[idxs[slot][c]],
                                      rows[c], srow).wait()

        def out_start(g, slot):
            pltpu.make_async_copy(outs[slot], out_hbm.at[pl.ds(g * T, T)],
                                  osems[slot]).start()

        def out_wait(slot):
            pltpu.make_async_copy(outs[slot], out_hbm.at[pl.ds(0, T)],
                                  osems[slot]).wait()

        def compute(slot):
            r0, r1, r2, r3 = rows
            o, wref = outs[slot], ws[slot]

            @pl.loop(0, T)
            def _bin(b):
                bvec = jnp.full((16,), b, jnp.int32)
                w0 = plsc.load_gather(wref, [bvec])
                w1 = plsc.load_gather(wref, [bvec + T])
                w2 = plsc.load_gather(wref, [bvec + 2 * T])
                w3 = plsc.load_gather(wref, [bvec + 3 * T])
                for j in range(0, C, 16):
                    s = pl.ds(j, 16)
                    o[b, s] = (w0 * r0[b, s] + w1 * r1[b, s]
                               + w2 * r2[b, s] + w3 * r3[b, s])

        idx_start(base, 0)
        w_start(base, 0)
        w_start(base + 1, 1)
        idx_start(base + 1, 1)

        def half(s, slot, do_prefetch, do_outwait):
            idx_wait(slot)
            gathers_start(slot)
            gathers_wait(slot)
            if do_prefetch:
                idx_start(base + s + 2, slot)
            if do_outwait:
                out_wait(slot)
            w_wait(slot)
            compute(slot)
            if do_prefetch:
                w_start(base + s + 2, slot)
            out_start(base + s, slot)

        half(0, 0, True, False)
        half(1, 1, True, False)

        @pl.loop(2, S - 2, step=2)
        def _pair(s):
            half(s, 0, True, True)
            half(s + 1, 1, True, True)

        half(S - 2, 0, False, True)
        half(S - 1, 1, False, True)

        out_wait(0)
        out_wait(1)

    return sc_kernel(table, idx_g, w_g)


def kernel(input, rois):
    B, C, H, W = input.shape
    n = rois.shape[0]
    assert n <= NPAD

    table = input.transpose(0, 2, 3, 1).reshape(B * H * W, C)
    rois_t = jnp.pad(rois.T, ((0, 0), (0, NPAD - n)))

    idx4, w4 = pl.pallas_call(
        functools.partial(_prep_body, n, H, W),
        out_shape=(
            jax.ShapeDtypeStruct((4, NBINS, NPAD), jnp.int32),
            jax.ShapeDtypeStruct((4, NBINS, NPAD), jnp.float32),
        ),
    )(rois_t)

    K0 = NBINS * NPAD
    K = ((K0 + 64 * T - 1) // (64 * T)) * (64 * T)
    G = K // T
    idx_flat = jnp.pad(idx4.reshape(4, K0), ((0, 0), (0, K - K0)))
    w_flat = jnp.pad(w4.reshape(4, K0), ((0, 0), (0, K - K0)))
    idx_g = idx_flat.reshape(4, G, T).transpose(1, 0, 2).reshape(G, 4 * T)
    w_g = w_flat.reshape(4, G, T).transpose(1, 0, 2).reshape(G, 4 * T)
    out_rows = _sc_pooled_rows(table, idx_g, w_g, C)
    out = out_rows[:K0].reshape(NBINS, NPAD, C)[:, :n]
    return out.transpose(1, 2, 0).reshape(n, C, POOLED, POOLED)


# speedup vs baseline: 1.2799x; 1.2799x over previous
import dataclasses
import functools
import math

import jax
import jax.numpy as jnp
from jax import lax
from jax.experimental import pallas as pl
from jax.experimental.pallas import tpu as pltpu
from jax.experimental.pallas import tpu_sc as plsc

POOLED = 7
NBINS = POOLED * POOLED
SCALE = 0.125
NPAD = 1024
T = 64


def _prep_body(n_real, H, W, rois_ref, idx_ref, w_ref):
    r = rois_ref[...]
    bidx = r[0:1, :].astype(jnp.int32)
    cx, cy = r[1:2, :], r[2:3, :]
    hh, ww = r[3:4, :], r[4:5, :]
    th = r[5:6, :] * (math.pi / 180.0)

    Sx = ww * (SCALE / POOLED)
    Sy = hh * (SCALE / POOLED)
    Al, Be = jnp.cos(th), jnp.sin(th)
    dx = dy = -POOLED / 2.0
    M00 = Al * Sx
    M01 = Be * Sy
    M02 = Al * Sx * dx + Be * Sy * dy + cx * SCALE
    M10 = -Be * Sx
    M11 = Al * Sy
    M12 = -Be * Sx * dx + Al * Sy * dy + cy * SCALE

    bi = lax.broadcasted_iota(jnp.int32, (NBINS, NPAD), 0)
    lane = lax.broadcasted_iota(jnp.int32, (NBINS, NPAD), 1)
    pwf = (bi % POOLED).astype(jnp.float32) + 0.5
    phf = (bi // POOLED).astype(jnp.float32) + 0.5
    Px = M00 * pwf + M01 * phf + M02
    Py = M10 * pwf + M11 * phf + M12

    vf = ((Px >= 0.0) & (Px <= W - 1.0) & (Py >= 0.0) & (Py <= H - 1.0)
          & (lane < n_real)).astype(jnp.float32)
    x0i = Px.astype(jnp.int32)
    y0i = Py.astype(jnp.int32)
    wx = Px - x0i.astype(jnp.float32)
    wy = Py - y0i.astype(jnp.float32)
    x0 = jnp.clip(x0i, 0, W - 1)
    y0 = jnp.clip(y0i, 0, H - 1)
    y1 = jnp.clip(y0i + 1, 0, H - 1)

    base = bidx * (H * W)
    idx_ref[0] = base + y0 * W + x0
    idx_ref[1] = base + y1 * W + x0
    wx1 = wx * (x0i < W - 1).astype(jnp.float32)
    w_ref[0] = (1.0 - wy) * (1.0 - wx) * vf
    w_ref[1] = (1.0 - wy) * wx1 * vf
    w_ref[2] = wy * (1.0 - wx) * vf
    w_ref[3] = wy * wx1 * vf


def _sc_pooled_rows(table2, idx_g, w_g, C):
    G = idx_g.shape[0]
    K = G * T
    mesh = plsc.VectorSubcoreMesh(core_axis_name="core", subcore_axis_name="subcore")

    cp = pltpu.CompilerParams()
    if "needs_layout_passes" in pltpu.CompilerParams.__dataclass_fields__:
        cp = dataclasses.replace(cp, needs_layout_passes=False)

    @functools.partial(
        pl.kernel,
        out_type=jax.ShapeDtypeStruct((K, C), jnp.float32),
        mesh=mesh,
        scratch_types=[pltpu.VMEM((T, 2 * C), jnp.float32) for _ in range(2)]
        + [pltpu.SemaphoreType.DMA],
        compiler_params=cp,
    )
    def sc_kernel(table_hbm, idx_hbm, w_hbm, out_hbm, p0, p1, sem):
        pairs = (p0, p1)

        def body(i_vmem, w_vmem, o_vmem):
            copies = [
                pltpu.async_copy(table_hbm.at[i_vmem.at[0, pl.ds(c * T, T)]],
                                 pairs[c], sem)
                for c in range(2)
            ]
            for copy in copies:
                copy.wait()

            @pl.loop(0, T)
            def _bin(b):
                bvec = jnp.full((16,), b, jnp.int32)
                zero = jnp.zeros((16,), jnp.int32)
                w00 = plsc.load_gather(w_vmem, [zero, bvec])
                w01 = plsc.load_gather(w_vmem, [zero, bvec + T])
                w10 = plsc.load_gather(w_vmem, [zero, bvec + 2 * T])
                w11 = plsc.load_gather(w_vmem, [zero, bvec + 3 * T])
                for j in range(0, C, 16):
                    s0 = pl.ds(j, 16)
                    s1 = pl.ds(C + j, 16)
                    o_vmem[b, s0] = (w00 * p0[b, s0] + w01 * p0[b, s1]
                                     + w10 * p1[b, s0] + w11 * p1[b, s1])

        pltpu.emit_pipeline(
            body,
            grid=(G,),
            in_specs=[
                pl.BlockSpec((1, 2 * T), lambda i: (i, 0)),
                pl.BlockSpec((1, 4 * T), lambda i: (i, 0)),
            ],
            out_specs=[pl.BlockSpec((T, C), lambda i: (i, 0))],
            core_axis_name=("core", "subcore"),
            dimension_semantics=(pltpu.PARALLEL,),
        )(idx_hbm, w_hbm, out_hbm)

    return sc_kernel(table2, idx_g, w_g)


def kernel(input, rois):
    B, C, H, W = input.shape
    n = rois.shape[0]
    assert n <= NPAD

    table = input.transpose(0, 2, 3, 1).reshape(B * H * W, C)
    tpad = jnp.concatenate([table, jnp.zeros((1, C), table.dtype)], axis=0)
    table2 = jnp.concatenate([tpad[:-1], tpad[1:]], axis=1)
    rois_t = jnp.pad(rois.T, ((0, 0), (0, NPAD - n)))

    idx2, w4 = pl.pallas_call(
        functools.partial(_prep_body, n, H, W),
        out_shape=(
            jax.ShapeDtypeStruct((2, NBINS, NPAD), jnp.int32),
            jax.ShapeDtypeStruct((4, NBINS, NPAD), jnp.float32),
        ),
    )(rois_t)

    K0 = NBINS * NPAD
    K = ((K0 + 32 * T - 1) // (32 * T)) * (32 * T)
    G = K // T
    idx_flat = jnp.pad(idx2.reshape(2, K0), ((0, 0), (0, K - K0)))
    w_flat = jnp.pad(w4.reshape(4, K0), ((0, 0), (0, K - K0)))
    idx_g = idx_flat.reshape(2, G, T).transpose(1, 0, 2).reshape(G, 2 * T)
    w_g = w_flat.reshape(4, G, T).transpose(1, 0, 2).reshape(G, 4 * T)
    out_rows = _sc_pooled_rows(table2, idx_g, w_g, C)
    out = out_rows[:K0].reshape(NBINS, NPAD, C)[:, :n]
    return out.transpose(1, 2, 0).reshape(n, C, POOLED, POOLED)


# speedup vs baseline: 1.3053x; 1.0198x over previous
import dataclasses
import functools
import math

import jax
import jax.numpy as jnp
from jax import lax
from jax.experimental import pallas as pl
from jax.experimental.pallas import tpu as pltpu
from jax.experimental.pallas import tpu_sc as plsc

POOLED = 7
NBINS = POOLED * POOLED
SCALE = 0.125
NPAD = 1024
T = 32


def _prep_body(n_real, H, W, rois_ref, idx_ref, w_ref):
    r = rois_ref[...]
    bidx = r[0:1, :].astype(jnp.int32)
    cx, cy = r[1:2, :], r[2:3, :]
    hh, ww = r[3:4, :], r[4:5, :]
    th = r[5:6, :] * (math.pi / 180.0)

    Sx = ww * (SCALE / POOLED)
    Sy = hh * (SCALE / POOLED)
    Al, Be = jnp.cos(th), jnp.sin(th)
    dx = dy = -POOLED / 2.0
    M00 = Al * Sx
    M01 = Be * Sy
    M02 = Al * Sx * dx + Be * Sy * dy + cx * SCALE
    M10 = -Be * Sx
    M11 = Al * Sy
    M12 = -Be * Sx * dx + Al * Sy * dy + cy * SCALE

    bi = lax.broadcasted_iota(jnp.int32, (NBINS, NPAD), 0)
    lane = lax.broadcasted_iota(jnp.int32, (NBINS, NPAD), 1)
    pwf = (bi % POOLED).astype(jnp.float32) + 0.5
    phf = (bi // POOLED).astype(jnp.float32) + 0.5
    Px = M00 * pwf + M01 * phf + M02
    Py = M10 * pwf + M11 * phf + M12

    vf = ((Px >= 0.0) & (Px <= W - 1.0) & (Py >= 0.0) & (Py <= H - 1.0)
          & (lane < n_real)).astype(jnp.float32)
    x0i = Px.astype(jnp.int32)
    y0i = Py.astype(jnp.int32)
    wx = Px - x0i.astype(jnp.float32)
    wy = Py - y0i.astype(jnp.float32)
    x0 = jnp.clip(x0i, 0, W - 1)
    x1 = jnp.clip(x0i + 1, 0, W - 1)
    y0 = jnp.clip(y0i, 0, H - 1)
    y1 = jnp.clip(y0i + 1, 0, H - 1)

    base = bidx * (H * W)
    idx_ref[0] = base + y0 * W + x0
    idx_ref[1] = base + y0 * W + x1
    idx_ref[2] = base + y1 * W + x0
    idx_ref[3] = base + y1 * W + x1
    w_ref[0] = (1.0 - wy) * (1.0 - wx) * vf
    w_ref[1] = (1.0 - wy) * wx * vf
    w_ref[2] = wy * (1.0 - wx) * vf
    w_ref[3] = wy * wx * vf


def _sc_pooled_rows(table, idx_g, w_g, C2):
    G = idx_g.shape[0]
    K = G * T
    mesh = plsc.VectorSubcoreMesh(core_axis_name="core", subcore_axis_name="subcore")

    cp = pltpu.CompilerParams()
    if "needs_layout_passes" in pltpu.CompilerParams.__dataclass_fields__:
        cp = dataclasses.replace(cp, needs_layout_passes=False)

    @functools.partial(
        pl.kernel,
        out_type=jax.ShapeDtypeStruct((K, C2), jnp.int32),
        mesh=mesh,
        scratch_types=[pltpu.VMEM((T, C2), jnp.int32) for _ in range(4)]
        + [pltpu.SemaphoreType.DMA],
        compiler_params=cp,
    )
    def sc_kernel(table_hbm, idx_hbm, w_hbm, out_hbm, r0, r1, r2, r3, sem):
        rows = (r0, r1, r2, r3)
        ILV = plsc.PackFormat.INTERLEAVED

        def body(i_vmem, w_vmem, o_vmem):
            copies = [
                pltpu.async_copy(table_hbm.at[i_vmem.at[0, pl.ds(c * T, T)]],
                                 rows[c], sem)
                for c in range(4)
            ]
            for copy in copies:
                copy.wait()

            @pl.loop(0, T)
            def _bin(b):
                bvec = jnp.full((16,), b, jnp.int32)
                zero = jnp.zeros((16,), jnp.int32)
                w0 = plsc.load_gather(w_vmem, [zero, bvec])
                w1 = plsc.load_gather(w_vmem, [zero, bvec + T])
                w2 = plsc.load_gather(w_vmem, [zero, bvec + 2 * T])
                w3 = plsc.load_gather(w_vmem, [zero, bvec + 3 * T])
                for j in range(0, C2, 16):
                    s = pl.ds(j, 16)
                    bf = jnp.bfloat16
                    a0, b0 = plsc.unpack(plsc.bitcast(r0[b, s], bf), format=ILV)
                    a1, b1 = plsc.unpack(plsc.bitcast(r1[b, s], bf), format=ILV)
                    a2, b2 = plsc.unpack(plsc.bitcast(r2[b, s], bf), format=ILV)
                    a3, b3 = plsc.unpack(plsc.bitcast(r3[b, s], bf), format=ILV)
                    oa = w0 * a0 + w1 * a1 + w2 * a2 + w3 * a3
                    ob = w0 * b0 + w1 * b1 + w2 * b2 + w3 * b3
                    o_vmem[b, s] = plsc.bitcast(
                        plsc.pack(oa, ob, format=ILV), jnp.int32)

        pltpu.emit_pipeline(
            body,
            grid=(G,),
            in_specs=[
                pl.BlockSpec((1, 4 * T), lambda i: (i, 0)),
                pl.BlockSpec((1, 4 * T), lambda i: (i, 0)),
            ],
            out_specs=[pl.BlockSpec((T, C2), lambda i: (i, 0))],
            core_axis_name=("core", "subcore"),
            dimension_semantics=(pltpu.PARALLEL,),
        )(idx_hbm, w_hbm, out_hbm)

    return sc_kernel(table, idx_g, w_g)


def kernel(input, rois):
    B, C, H, W = input.shape
    n = rois.shape[0]
    assert n <= NPAD

    table = input.transpose(0, 2, 3, 1).reshape(B * H * W, C)
    table = lax.bitcast_convert_type(
        table.astype(jnp.bfloat16).reshape(B * H * W, C // 2, 2), jnp.int32)
    rois_t = jnp.pad(rois.T, ((0, 0), (0, NPAD - n)))

    idx4, w4 = pl.pallas_call(
        functools.partial(_prep_body, n, H, W),
        out_shape=(
            jax.ShapeDtypeStruct((4, NBINS, NPAD), jnp.int32),
            jax.ShapeDtypeStruct((4, NBINS, NPAD), jnp.float32),
        ),
    )(rois_t)

    K0 = NBINS * NPAD
    K = ((K0 + 32 * T - 1) // (32 * T)) * (32 * T)
    G = K // T
    idx_flat = jnp.pad(idx4.reshape(4, K0), ((0, 0), (0, K - K0)))
    w_flat = jnp.pad(w4.reshape(4, K0), ((0, 0), (0, K - K0)))
    idx_g = idx_flat.reshape(4, G, T).transpose(1, 0, 2).reshape(G, 4 * T)
    w_g = w_flat.reshape(4, G, T).transpose(1, 0, 2).reshape(G, 4 * T)
    out_rows = _sc_pooled_rows(table, idx_g, w_g, C // 2)
    out_bf = lax.bitcast_convert_type(out_rows, jnp.bfloat16).reshape(K, C)
    out = out_bf[:K0].reshape(NBINS, NPAD, C)[:, :n]
    return out.transpose(1, 2, 0).astype(jnp.float32).reshape(n, C, POOLED, POOLED)


# speedup vs baseline: 1.8818x; 1.4417x over previous
import dataclasses
import functools
import math

import jax
import jax.numpy as jnp
from jax import lax
from jax.experimental import pallas as pl
from jax.experimental.pallas import tpu as pltpu
from jax.experimental.pallas import tpu_sc as plsc

POOLED = 7
NBINS = POOLED * POOLED
SCALE = 0.125
NPAD = 1024
T = 32


def _prep_body(n_real, H, W, rois_ref, idx_ref, w_ref):
    r = rois_ref[...]
    bidx = r[0:1, :].astype(jnp.int32)
    cx, cy = r[1:2, :], r[2:3, :]
    hh, ww = r[3:4, :], r[4:5, :]
    th = r[5:6, :] * (math.pi / 180.0)

    Sx = ww * (SCALE / POOLED)
    Sy = hh * (SCALE / POOLED)
    Al, Be = jnp.cos(th), jnp.sin(th)
    dx = dy = -POOLED / 2.0
    M00 = Al * Sx
    M01 = Be * Sy
    M02 = Al * Sx * dx + Be * Sy * dy + cx * SCALE
    M10 = -Be * Sx
    M11 = Al * Sy
    M12 = -Be * Sx * dx + Al * Sy * dy + cy * SCALE

    bi = lax.broadcasted_iota(jnp.int32, (NBINS, NPAD), 0)
    lane = lax.broadcasted_iota(jnp.int32, (NBINS, NPAD), 1)
    pwf = (bi % POOLED).astype(jnp.float32) + 0.5
    phf = (bi // POOLED).astype(jnp.float32) + 0.5
    Px = M00 * pwf + M01 * phf + M02
    Py = M10 * pwf + M11 * phf + M12

    vf = ((Px >= 0.0) & (Px <= W - 1.0) & (Py >= 0.0) & (Py <= H - 1.0)
          & (lane < n_real)).astype(jnp.float32)
    x0i = Px.astype(jnp.int32)
    y0i = Py.astype(jnp.int32)
    wx = Px - x0i.astype(jnp.float32)
    wy = Py - y0i.astype(jnp.float32)
    x0 = jnp.clip(x0i, 0, W - 1)
    x1 = jnp.clip(x0i + 1, 0, W - 1)
    y0 = jnp.clip(y0i, 0, H - 1)
    y1 = jnp.clip(y0i + 1, 0, H - 1)

    base = bidx * (H * W)
    idx_ref[0] = base + y0 * W + x0
    idx_ref[1] = base + y0 * W + x1
    idx_ref[2] = base + y1 * W + x0
    idx_ref[3] = base + y1 * W + x1
    w_ref[0] = (1.0 - wy) * (1.0 - wx) * vf
    w_ref[1] = (1.0 - wy) * wx * vf
    w_ref[2] = wy * (1.0 - wx) * vf
    w_ref[3] = wy * wx * vf


def _sc_pooled_rows(table, idx_g, w_g, C):
    G = idx_g.shape[0]
    K = G * T
    mesh = plsc.VectorSubcoreMesh(core_axis_name="core", subcore_axis_name="subcore")

    cp = pltpu.CompilerParams()
    if "needs_layout_passes" in pltpu.CompilerParams.__dataclass_fields__:
        cp = dataclasses.replace(cp, needs_layout_passes=False)

    @functools.partial(
        pl.kernel,
        out_type=jax.ShapeDtypeStruct((K, C), jnp.float32),
        mesh=mesh,
        scratch_types=[pltpu.VMEM((T, C), jnp.float32) for _ in range(4)]
        + [pltpu.SemaphoreType.DMA],
        compiler_params=cp,
    )
    def sc_kernel(table_hbm, idx_hbm, w_hbm, out_hbm, r0, r1, r2, r3, sem):
        rows = (r0, r1, r2, r3)

        def body(i_vmem, w_vmem, o_vmem):
            copies = [
                pltpu.async_copy(table_hbm.at[i_vmem.at[0, pl.ds(c * T, T)]],
                                 rows[c], sem)
                for c in range(4)
            ]
            for copy in copies:
                copy.wait()

            @pl.loop(0, T)
            def _bin(b):
                bvec = jnp.full((16,), b, jnp.int32)
                zero = jnp.zeros((16,), jnp.int32)
                w0 = plsc.load_gather(w_vmem, [zero, bvec])
                w1 = plsc.load_gather(w_vmem, [zero, bvec + T])
                w2 = plsc.load_gather(w_vmem, [zero, bvec + 2 * T])
                w3 = plsc.load_gather(w_vmem, [zero, bvec + 3 * T])
                for j in range(0, C, 16):
                    s = pl.ds(j, 16)
                    o_vmem[b, s] = (w0 * r0[b, s] + w1 * r1[b, s]
                                    + w2 * r2[b, s] + w3 * r3[b, s])

        pltpu.emit_pipeline(
            body,
            grid=(G,),
            in_specs=[
                pl.BlockSpec((1, 4 * T), lambda i: (i, 0)),
                pl.BlockSpec((1, 4 * T), lambda i: (i, 0)),
            ],
            out_specs=[pl.BlockSpec((T, C), lambda i: (i, 0))],
            core_axis_name=("core", "subcore"),
            dimension_semantics=(pltpu.PARALLEL,),
        )(idx_hbm, w_hbm, out_hbm)

    return sc_kernel(table, idx_g, w_g)


def kernel(input, rois):
    B, C, H, W = input.shape
    n = rois.shape[0]
    assert n <= NPAD

    table = input.transpose(0, 2, 3, 1).reshape(B * H * W, C)
    rois_t = jnp.pad(rois.T, ((0, 0), (0, NPAD - n)))

    idx4, w4 = pl.pallas_call(
        functools.partial(_prep_body, n, H, W),
        out_shape=(
            jax.ShapeDtypeStruct((4, NBINS, NPAD), jnp.int32),
            jax.ShapeDtypeStruct((4, NBINS, NPAD), jnp.float32),
        ),
    )(rois_t)

    K0 = NBINS * NPAD
    K = ((K0 + 32 * T - 1) // (32 * T)) * (32 * T)
    G = K // T
    idx_flat = jnp.pad(idx4.reshape(4, K0), ((0, 0), (0, K - K0)))
    w_flat = jnp.pad(w4.reshape(4, K0), ((0, 0), (0, K - K0)))
    idx_g = idx_flat.reshape(4, G, T).transpose(1, 0, 2).reshape(G, 4 * T)
    w_g = w_flat.reshape(4, G, T).transpose(1, 0, 2).reshape(G, 4 * T)
    out_rows = _sc_pooled_rows(table, idx_g, w_g, C)
    out = out_rows[:K0].reshape(NBINS, NPAD, C)[:, :n]
    return out.transpose(1, 2, 0).reshape(n, C, POOLED, POOLED)


# speedup vs baseline: 1.8861x; 1.0023x over previous
import dataclasses
import functools
import math

import jax
import jax.numpy as jnp
from jax import lax
from jax.experimental import pallas as pl
from jax.experimental.pallas import tpu as pltpu
from jax.experimental.pallas import tpu_sc as plsc

POOLED = 7
NBINS = POOLED * POOLED
SCALE = 0.125
NPAD = 1024
T = 32


def _prep_body(n_real, H, W, rois_ref, idx_ref, w_ref):
    r = rois_ref[...]
    bidx = r[0:1, :].astype(jnp.int32)
    cx, cy = r[1:2, :], r[2:3, :]
    hh, ww = r[3:4, :], r[4:5, :]
    th = r[5:6, :] * (math.pi / 180.0)

    Sx = ww * (SCALE / POOLED)
    Sy = hh * (SCALE / POOLED)
    Al, Be = jnp.cos(th), jnp.sin(th)
    dx = dy = -POOLED / 2.0
    M00 = Al * Sx
    M01 = Be * Sy
    M02 = Al * Sx * dx + Be * Sy * dy + cx * SCALE
    M10 = -Be * Sx
    M11 = Al * Sy
    M12 = -Be * Sx * dx + Al * Sy * dy + cy * SCALE

    bi = lax.broadcasted_iota(jnp.int32, (NBINS, NPAD), 0)
    lane = lax.broadcasted_iota(jnp.int32, (NBINS, NPAD), 1)
    pwf = (bi % POOLED).astype(jnp.float32) + 0.5
    phf = (bi // POOLED).astype(jnp.float32) + 0.5
    Px = M00 * pwf + M01 * phf + M02
    Py = M10 * pwf + M11 * phf + M12

    vf = ((Px >= 0.0) & (Px <= W - 1.0) & (Py >= 0.0) & (Py <= H - 1.0)
          & (lane < n_real)).astype(jnp.float32)
    x0i = Px.astype(jnp.int32)
    y0i = Py.astype(jnp.int32)
    wx = Px - x0i.astype(jnp.float32)
    wy = Py - y0i.astype(jnp.float32)
    x0 = jnp.clip(x0i, 0, W - 1)
    x1 = jnp.clip(x0i + 1, 0, W - 1)
    y0 = jnp.clip(y0i, 0, H - 1)
    y1 = jnp.clip(y0i + 1, 0, H - 1)

    base = bidx * (H * W)
    idx_ref[0] = base + y0 * W + x0
    idx_ref[1] = base + y0 * W + x1
    idx_ref[2] = base + y1 * W + x0
    idx_ref[3] = base + y1 * W + x1
    w_ref[0] = (1.0 - wy) * (1.0 - wx) * vf
    w_ref[1] = (1.0 - wy) * wx * vf
    w_ref[2] = wy * (1.0 - wx) * vf
    w_ref[3] = wy * wx * vf


def _sc_pooled_rows(table, idx_g, w_g, C):
    G = idx_g.shape[0]
    K = G * T
    mesh = plsc.VectorSubcoreMesh(core_axis_name="core", subcore_axis_name="subcore")

    cp = pltpu.CompilerParams()
    if "needs_layout_passes" in pltpu.CompilerParams.__dataclass_fields__:
        cp = dataclasses.replace(cp, needs_layout_passes=False)

    @functools.partial(
        pl.kernel,
        out_type=jax.ShapeDtypeStruct((K, C), jnp.float32),
        mesh=mesh,
        scratch_types=[pltpu.VMEM((T, C), jnp.float32) for _ in range(4)]
        + [pltpu.SemaphoreType.DMA],
        compiler_params=cp,
    )
    def sc_kernel(table_hbm, idx_hbm, w_hbm, out_hbm, r0, r1, r2, r3, sem):
        rows = (r0, r1, r2, r3)

        def body(i_vmem, w_vmem, o_vmem):
            copies = [
                pltpu.async_copy(table_hbm.at[i_vmem.at[0, pl.ds(c * T, T)]],
                                 rows[c], sem)
                for c in range(4)
            ]
            for copy in copies:
                copy.wait()

            @pl.loop(0, T, unroll=4)
            def _bin(b):
                bvec = jnp.full((16,), b, jnp.int32)
                zero = jnp.zeros((16,), jnp.int32)
                w0 = plsc.load_gather(w_vmem, [zero, bvec])
                w1 = plsc.load_gather(w_vmem, [zero, bvec + T])
                w2 = plsc.load_gather(w_vmem, [zero, bvec + 2 * T])
                w3 = plsc.load_gather(w_vmem, [zero, bvec + 3 * T])
                for j in range(0, C, 16):
                    s = pl.ds(j, 16)
                    o_vmem[b, s] = (w0 * r0[b, s] + w1 * r1[b, s]
                                    + w2 * r2[b, s] + w3 * r3[b, s])

        pltpu.emit_pipeline(
            body,
            grid=(G,),
            in_specs=[
                pl.BlockSpec((1, 4 * T), lambda i: (i, 0)),
                pl.BlockSpec((1, 4 * T), lambda i: (i, 0)),
            ],
            out_specs=[pl.BlockSpec((T, C), lambda i: (i, 0))],
            core_axis_name=("core", "subcore"),
            dimension_semantics=(pltpu.PARALLEL,),
        )(idx_hbm, w_hbm, out_hbm)

    return sc_kernel(table, idx_g, w_g)


def kernel(input, rois):
    B, C, H, W = input.shape
    n = rois.shape[0]
    assert n <= NPAD

    table = input.transpose(0, 2, 3, 1).reshape(B * H * W, C)
    rois_t = jnp.pad(rois.T, ((0, 0), (0, NPAD - n)))

    idx4, w4 = pl.pallas_call(
        functools.partial(_prep_body, n, H, W),
        out_shape=(
            jax.ShapeDtypeStruct((4, NBINS, NPAD), jnp.int32),
            jax.ShapeDtypeStruct((4, NBINS, NPAD), jnp.float32),
        ),
    )(rois_t)

    K0 = NBINS * NPAD
    K = ((K0 + 32 * T - 1) // (32 * T)) * (32 * T)
    G = K // T
    idx_flat = jnp.pad(idx4.reshape(4, K0), ((0, 0), (0, K - K0)))
    w_flat = jnp.pad(w4.reshape(4, K0), ((0, 0), (0, K - K0)))
    idx_g = idx_flat.reshape(4, G, T).transpose(1, 0, 2).reshape(G, 4 * T)
    w_g = w_flat.reshape(4, G, T).transpose(1, 0, 2).reshape(G, 4 * T)
    out_rows = _sc_pooled_rows(table, idx_g, w_g, C)
    out = out_rows[:K0].reshape(NBINS, NPAD, C)[:, :n]
    return out.transpose(1, 2, 0).reshape(n, C, POOLED, POOLED)
